# baseline (speedup 1.0000x reference)
import functools

import jax
import jax.numpy as jnp
from jax import lax
from jax.experimental import pallas as pl
from jax.experimental.pallas import tpu as pltpu
from jax.experimental.pallas import tpu_sc as plsc

N = 8192
E = 32768
G = 256
DN = 128
DE = 16
H = 64
DOUT = 16

NC = 2
NS = 16
NW = NC * NS
EW = E // NW
CH = 128
NCH = EW // CH
NROW = N // NS

BM = 256
BN = 1024


def _lrelu(v):
    return jnp.where(v >= 0, v, 0.01 * v)


def _nfc_body(x_ref, w_ref, b_ref, o_ref):
    o_ref[...] = _lrelu(
        jnp.dot(x_ref[...], w_ref[...], preferred_element_type=jnp.float32)
        + b_ref[0:1]
    )


def _nfc(x, W, b):
    n, d = x.shape
    h = W.shape[1]
    bm = 2048
    return pl.pallas_call(
        _nfc_body,
        grid=(n // bm,),
        in_specs=[
            pl.BlockSpec((bm, d), lambda i: (i, 0)),
            pl.BlockSpec((d, h), lambda i: (0, 0)),
            pl.BlockSpec((8, h), lambda i: (0, 0)),
        ],
        out_specs=pl.BlockSpec((bm, h), lambda i: (i, 0)),
        out_shape=jax.ShapeDtypeStruct((n, h), jnp.float32),
    )(x, W, jnp.broadcast_to(b, (8, h)))


def _msg_body(ea_ref, hxs_ref, w1_ref, b1_ref, w2_ref, b2_ref, o_ref):
    h1 = jnp.maximum(
        jnp.dot(ea_ref[...], w1_ref[...], preferred_element_type=jnp.float32)
        + b1_ref[0:1],
        0.0,
    )
    theta = (
        jnp.dot(h1, w2_ref[...], preferred_element_type=jnp.float32)
        + b2_ref[0:1]
    )
    hxs = hxs_ref[...]
    acc = hxs[:, 0:1] * theta[:, 0:H]
    for i in range(1, H):
        acc = acc + hxs[:, i : i + 1] * theta[:, i * H : (i + 1) * H]
    o_ref[...] = acc


def _msg(edge_attr, hxs, W1, b1, W2, b2):
    return pl.pallas_call(
        _msg_body,
        grid=(E // BM,),
        in_specs=[
            pl.BlockSpec((BM, DE), lambda i: (i, 0)),
            pl.BlockSpec((BM, H), lambda i: (i, 0)),
            pl.BlockSpec((DE, H), lambda i: (0, 0)),
            pl.BlockSpec((8, H), lambda i: (0, 0)),
            pl.BlockSpec((H, H * H), lambda i: (0, 0)),
            pl.BlockSpec((8, H * H), lambda i: (0, 0)),
        ],
        out_specs=pl.BlockSpec((BM, H), lambda i: (i, 0)),
        out_shape=jax.ShapeDtypeStruct((E, H), jnp.float32),
    )(
        edge_attr,
        hxs,
        W1,
        jnp.broadcast_to(b1, (8, H)),
        W2,
        jnp.broadcast_to(b2, (8, H * H)),
    )


def _gather(table, idx3):
    mesh = plsc.VectorSubcoreMesh(core_axis_name="c", subcore_axis_name="s")

    @functools.partial(
        pl.kernel,
        mesh=mesh,
        out_type=jax.ShapeDtypeStruct((E, H), jnp.float32),
        scratch_types=[
            pltpu.VMEM((NCH, CH), jnp.int32),
            pltpu.VMEM((EW, H), jnp.float32),
            pltpu.SemaphoreType.DMA,
        ],
        compiler_params=pltpu.CompilerParams(use_tc_tiling_on_sc=False),
    )
    def k(table_hbm, idx_hbm, out_hbm, idx_v, rows_v, sem):
        cid = lax.axis_index("c")
        sid = lax.axis_index("s")
        wid = cid * NS + sid
        pltpu.sync_copy(idx_hbm.at[wid], idx_v)
        cps = []
        for j in range(NCH):
            cps.append(
                pltpu.async_copy(
                    table_hbm.at[idx_v.at[j]],
                    rows_v.at[pl.ds(j * CH, CH)],
                    sem,
                )
            )
        for c in cps:
            c.wait()
        pltpu.sync_copy(rows_v, out_hbm.at[pl.ds(wid * EW, EW)])

    return k(table, idx3)


def _scatter_add(msg, dst3, zrow):
    mesh = plsc.VectorSubcoreMesh(core_axis_name="c", subcore_axis_name="s")

    @functools.partial(
        pl.kernel,
        mesh=mesh,
        out_type=jax.ShapeDtypeStruct((NC * N, H), jnp.float32),
        scratch_types=[
            pltpu.VMEM((NCH, CH), jnp.int32),
            pltpu.VMEM((EW, H), jnp.float32),
            pltpu.VMEM_SHARED((N, H), jnp.float32),
        ],
        compiler_params=pltpu.CompilerParams(use_tc_tiling_on_sc=False),
    )
    def k(msg_hbm, dst_hbm, z_hbm, out_hbm, idx_v, rows_v, acc_sh):
        cid = lax.axis_index("c")
        sid = lax.axis_index("s")
        wid = cid * NS + sid
        pltpu.sync_copy(z_hbm, acc_sh.at[pl.ds(sid * NROW, NROW)])
        pltpu.sync_copy(dst_hbm.at[wid], idx_v)
        pltpu.sync_copy(msg_hbm.at[pl.ds(wid * EW, EW)], rows_v)
        plsc.subcore_barrier()
        for j in range(NCH):
            pltpu.sync_copy(
                rows_v.at[pl.ds(j * CH, CH)],
                acc_sh.at[idx_v.at[j]],
                add=True,
            )
        plsc.subcore_barrier()
        pltpu.sync_copy(
            acc_sh.at[pl.ds(sid * NROW, NROW)],
            out_hbm.at[pl.ds(cid * N + sid * NROW, NROW)],
        )

    return k(msg, dst3, zrow)


def _combine_body(p0_ref, p1_ref, hx_ref, w_ref, b_ref, o_ref):
    o_ref[...] = _lrelu(
        p0_ref[...]
        + p1_ref[...]
        + jnp.dot(hx_ref[...], w_ref[...], preferred_element_type=jnp.float32)
        + b_ref[0:1]
    )


def _combine(parts, hx, rootW, bias):
    bm = 2048
    nb = N // bm
    return pl.pallas_call(
        _combine_body,
        grid=(nb,),
        in_specs=[
            pl.BlockSpec((bm, H), lambda i: (i, 0)),
            pl.BlockSpec((bm, H), lambda i: (i + nb, 0)),
            pl.BlockSpec((bm, H), lambda i: (i, 0)),
            pl.BlockSpec((H, H), lambda i: (0, 0)),
            pl.BlockSpec((8, H), lambda i: (0, 0)),
        ],
        out_specs=pl.BlockSpec((bm, H), lambda i: (i, 0)),
        out_shape=jax.ShapeDtypeStruct((N, H), jnp.float32),
    )(parts, parts, hx, rootW, jnp.broadcast_to(bias, (8, H)))


def _final_body(
    p0_ref, p1_ref, hx_ref, root_ref, gb_ref, batch_ref,
    f1w_ref, f1b_ref, f2w_ref, f2b_ref, o_ref, hg_ref
):
    i = pl.program_id(0)
    hx2 = _lrelu(
        p0_ref[...]
        + p1_ref[...]
        + jnp.dot(hx_ref[...], root_ref[...], preferred_element_type=jnp.float32)
        + gb_ref[0:1]
    )
    b = batch_ref[0]
    gid = lax.broadcasted_iota(jnp.int32, (G, BN), 0)
    m = (b == gid).astype(jnp.float32)
    part = jnp.dot(m, hx2, preferred_element_type=jnp.float32)

    @pl.when(i == 0)
    def _():
        hg_ref[...] = part

    @pl.when(i > 0)
    def _():
        hg_ref[...] = hg_ref[...] + part

    @pl.when(i == pl.num_programs(0) - 1)
    def _():
        hg = _lrelu(
            jnp.dot(hg_ref[...], f1w_ref[...], preferred_element_type=jnp.float32)
            + f1b_ref[0:1]
        )
        o_ref[...] = (
            jnp.dot(hg, f2w_ref[...], preferred_element_type=jnp.float32)
            + f2b_ref[0:1]
        )


def _final(parts, hx, rootW, bias, batch3, f1w, f1b, f2w, f2b):
    nb = N // BN
    return pl.pallas_call(
        _final_body,
        grid=(nb,),
        in_specs=[
            pl.BlockSpec((BN, H), lambda i: (i, 0)),
            pl.BlockSpec((BN, H), lambda i: (i + nb, 0)),
            pl.BlockSpec((BN, H), lambda i: (i, 0)),
            pl.BlockSpec((H, H), lambda i: (0, 0)),
            pl.BlockSpec((8, H), lambda i: (0, 0)),
            pl.BlockSpec((1, 1, BN), lambda i: (i, 0, 0)),
            pl.BlockSpec((H, 32), lambda i: (0, 0)),
            pl.BlockSpec((8, 32), lambda i: (0, 0)),
            pl.BlockSpec((32, DOUT), lambda i: (0, 0)),
            pl.BlockSpec((8, DOUT), lambda i: (0, 0)),
        ],
        out_specs=pl.BlockSpec((G, DOUT), lambda i: (0, 0)),
        out_shape=jax.ShapeDtypeStruct((G, DOUT), jnp.float32),
        scratch_shapes=[pltpu.VMEM((G, H), jnp.float32)],
    )(
        parts, parts, hx, rootW, jnp.broadcast_to(bias, (8, H)), batch3,
        f1w, jnp.broadcast_to(f1b, (8, 32)),
        f2w, jnp.broadcast_to(f2b, (8, DOUT)),
    )


def kernel(x, edge_index, edge_attr, batch, nfc_W, nfc_b,
           efc1_W1, efc1_b1, efc1_W2, efc1_b2, gc1_root, gc1_bias,
           efc2_W1, efc2_b1, efc2_W2, efc2_b2, gc2_root, gc2_bias,
           fc1_W, fc1_b, fc2_W, fc2_b):
    src3 = edge_index[0].astype(jnp.int32).reshape(NW, NCH, CH)
    dst3 = edge_index[1].astype(jnp.int32).reshape(NW, NCH, CH)
    zrow = jnp.zeros((NROW, H), jnp.float32)
    batch3 = batch.astype(jnp.int32).reshape(N // BN, 1, BN)

    hx0 = _nfc(x, nfc_W, nfc_b)

    hxs1 = _gather(hx0, src3)
    msg1 = _msg(edge_attr, hxs1, efc1_W1, efc1_b1, efc1_W2, efc1_b2)
    parts1 = _scatter_add(msg1, dst3, zrow)
    hx1 = _combine(parts1, hx0, gc1_root, gc1_bias)

    hxs2 = _gather(hx1, src3)
    msg2 = _msg(edge_attr, hxs2, efc2_W1, efc2_b1, efc2_W2, efc2_b2)
    parts2 = _scatter_add(msg2, dst3, zrow)
    return _final(parts2, hx1, gc2_root, gc2_bias, batch3,
                  fc1_W, fc1_b, fc2_W, fc2_b)


# speedup vs baseline: 3.8383x; 3.8383x over previous
import functools

import jax
import jax.numpy as jnp
from jax import lax
from jax.experimental import pallas as pl
from jax.experimental.pallas import tpu as pltpu
from jax.experimental.pallas import tpu_sc as plsc

N = 8192
E = 32768
G = 256
DN = 128
DE = 16
H = 64
DOUT = 16

NC = 2
NS = 16
NW = NC * NS
EW = E // NW
CH = 128
NCH = EW // CH
NROW = N // NS

BM = 1024
BN = 1024


def _lrelu(v):
    return jnp.where(v >= 0, v, 0.01 * v)


def _nfc_body(x_ref, w_ref, b_ref, o_ref):
    o_ref[...] = _lrelu(
        jnp.dot(x_ref[...], w_ref[...], preferred_element_type=jnp.float32)
        + b_ref[0:1]
    )


def _nfc(x, W, b):
    n, d = x.shape
    h = W.shape[1]
    bm = 2048
    return pl.pallas_call(
        _nfc_body,
        grid=(n // bm,),
        in_specs=[
            pl.BlockSpec((bm, d), lambda i: (i, 0)),
            pl.BlockSpec((d, h), lambda i: (0, 0)),
            pl.BlockSpec((8, h), lambda i: (0, 0)),
        ],
        out_specs=pl.BlockSpec((bm, h), lambda i: (i, 0)),
        out_shape=jax.ShapeDtypeStruct((n, h), jnp.float32),
    )(x, W, jnp.broadcast_to(b, (8, h)))


def _msg_body(eat_ref, hxs_ref, w1t_ref, b1c_ref, w2t_ref, b2rt_ref, o_ref):
    h1t = jnp.maximum(
        jnp.dot(w1t_ref[...], eat_ref[...], preferred_element_type=jnp.float32)
        + b1c_ref[:, 0:1],
        0.0,
    )
    h1tb = h1t.astype(jnp.bfloat16)
    hxst = hxs_ref[...].T
    acc0 = jnp.dot(b2rt_ref[...], hxst, preferred_element_type=jnp.float32)
    acc1 = jnp.zeros_like(acc0)
    NG = 8
    IPG = H // NG
    ROWS = IPG * H
    for g in range(NG):
        tg = jnp.dot(
            w2t_ref[g * ROWS : (g + 1) * ROWS, :], h1tb,
            preferred_element_type=jnp.float32,
        )
        for i2 in range(IPG):
            i = g * IPG + i2
            term = hxst[i : i + 1, :] * tg[i2 * H : (i2 + 1) * H, :]
            if i2 % 2 == 0:
                acc0 = acc0 + term
            else:
                acc1 = acc1 + term
    o_ref[...] = (acc0 + acc1).T


def _msg(edge_attr_t, hxs, W1, b1, W2, b2):
    return pl.pallas_call(
        _msg_body,
        grid=(E // BM,),
        in_specs=[
            pl.BlockSpec((DE, BM), lambda i: (0, i)),
            pl.BlockSpec((BM, H), lambda i: (i, 0)),
            pl.BlockSpec((H, DE), lambda i: (0, 0)),
            pl.BlockSpec((H, 8), lambda i: (0, 0)),
            pl.BlockSpec((H * H, H), lambda i: (0, 0)),
            pl.BlockSpec((H, H), lambda i: (0, 0)),
        ],
        out_specs=pl.BlockSpec((BM, H), lambda i: (i, 0)),
        out_shape=jax.ShapeDtypeStruct((E, H), jnp.float32),
    )(
        edge_attr_t,
        hxs,
        W1.T,
        jnp.broadcast_to(b1[:, None], (H, 8)),
        W2.T.astype(jnp.bfloat16),
        b2.reshape(H, H).T,
    )


def _gather(table, idx3):
    mesh = plsc.VectorSubcoreMesh(core_axis_name="c", subcore_axis_name="s")

    @functools.partial(
        pl.kernel,
        mesh=mesh,
        out_type=jax.ShapeDtypeStruct((E, H), jnp.float32),
        scratch_types=[
            pltpu.VMEM((NCH, CH), jnp.int32),
            pltpu.VMEM((EW, H), jnp.float32),
            pltpu.SemaphoreType.DMA,
        ],
        compiler_params=pltpu.CompilerParams(use_tc_tiling_on_sc=False),
    )
    def k(table_hbm, idx_hbm, out_hbm, idx_v, rows_v, sem):
        cid = lax.axis_index("c")
        sid = lax.axis_index("s")
        wid = cid * NS + sid
        pltpu.sync_copy(idx_hbm.at[wid], idx_v)
        cps = []
        for j in range(NCH):
            cps.append(
                pltpu.async_copy(
                    table_hbm.at[idx_v.at[j]],
                    rows_v.at[pl.ds(j * CH, CH)],
                    sem,
                )
            )
        for c in cps:
            c.wait()
        pltpu.sync_copy(rows_v, out_hbm.at[pl.ds(wid * EW, EW)])

    return k(table, idx3)


def _scatter_add(msg, dst3, zrow):
    mesh = plsc.VectorSubcoreMesh(core_axis_name="c", subcore_axis_name="s")

    @functools.partial(
        pl.kernel,
        mesh=mesh,
        out_type=jax.ShapeDtypeStruct((NC * N, H), jnp.float32),
        scratch_types=[
            pltpu.VMEM((NCH, CH), jnp.int32),
            pltpu.VMEM((EW, H), jnp.float32),
            pltpu.VMEM_SHARED((N, H), jnp.float32),
        ],
        compiler_params=pltpu.CompilerParams(use_tc_tiling_on_sc=False),
    )
    def k(msg_hbm, dst_hbm, z_hbm, out_hbm, idx_v, rows_v, acc_sh):
        cid = lax.axis_index("c")
        sid = lax.axis_index("s")
        wid = cid * NS + sid
        pltpu.sync_copy(z_hbm, acc_sh.at[pl.ds(sid * NROW, NROW)])
        pltpu.sync_copy(dst_hbm.at[wid], idx_v)
        pltpu.sync_copy(msg_hbm.at[pl.ds(wid * EW, EW)], rows_v)
        plsc.subcore_barrier()
        for j in range(NCH):
            pltpu.sync_copy(
                rows_v.at[pl.ds(j * CH, CH)],
                acc_sh.at[idx_v.at[j]],
                add=True,
            )
        plsc.subcore_barrier()
        pltpu.sync_copy(
            acc_sh.at[pl.ds(sid * NROW, NROW)],
            out_hbm.at[pl.ds(cid * N + sid * NROW, NROW)],
        )

    return k(msg, dst3, zrow)


def _combine_body(p0_ref, p1_ref, hx_ref, w_ref, b_ref, o_ref):
    o_ref[...] = _lrelu(
        p0_ref[...]
        + p1_ref[...]
        + jnp.dot(hx_ref[...], w_ref[...], preferred_element_type=jnp.float32)
        + b_ref[0:1]
    )


def _combine(parts, hx, rootW, bias):
    bm = 2048
    nb = N // bm
    return pl.pallas_call(
        _combine_body,
        grid=(nb,),
        in_specs=[
            pl.BlockSpec((bm, H), lambda i: (i, 0)),
            pl.BlockSpec((bm, H), lambda i: (i + nb, 0)),
            pl.BlockSpec((bm, H), lambda i: (i, 0)),
            pl.BlockSpec((H, H), lambda i: (0, 0)),
            pl.BlockSpec((8, H), lambda i: (0, 0)),
        ],
        out_specs=pl.BlockSpec((bm, H), lambda i: (i, 0)),
        out_shape=jax.ShapeDtypeStruct((N, H), jnp.float32),
    )(parts, parts, hx, rootW, jnp.broadcast_to(bias, (8, H)))


def _final_body(
    p0_ref, p1_ref, hx_ref, root_ref, gb_ref, batch_ref,
    f1w_ref, f1b_ref, f2w_ref, f2b_ref, o_ref, hg_ref
):
    i = pl.program_id(0)
    hx2 = _lrelu(
        p0_ref[...]
        + p1_ref[...]
        + jnp.dot(hx_ref[...], root_ref[...], preferred_element_type=jnp.float32)
        + gb_ref[0:1]
    )
    b = batch_ref[0]
    gid = lax.broadcasted_iota(jnp.int32, (G, BN), 0)
    m = (b == gid).astype(jnp.float32)
    part = jnp.dot(m, hx2, preferred_element_type=jnp.float32)

    @pl.when(i == 0)
    def _():
        hg_ref[...] = part

    @pl.when(i > 0)
    def _():
        hg_ref[...] = hg_ref[...] + part

    @pl.when(i == pl.num_programs(0) - 1)
    def _():
        hg = _lrelu(
            jnp.dot(hg_ref[...], f1w_ref[...], preferred_element_type=jnp.float32)
            + f1b_ref[0:1]
        )
        o_ref[...] = (
            jnp.dot(hg, f2w_ref[...], preferred_element_type=jnp.float32)
            + f2b_ref[0:1]
        )


def _final(parts, hx, rootW, bias, batch3, f1w, f1b, f2w, f2b):
    nb = N // BN
    return pl.pallas_call(
        _final_body,
        grid=(nb,),
        in_specs=[
            pl.BlockSpec((BN, H), lambda i: (i, 0)),
            pl.BlockSpec((BN, H), lambda i: (i + nb, 0)),
            pl.BlockSpec((BN, H), lambda i: (i, 0)),
            pl.BlockSpec((H, H), lambda i: (0, 0)),
            pl.BlockSpec((8, H), lambda i: (0, 0)),
            pl.BlockSpec((1, 1, BN), lambda i: (i, 0, 0)),
            pl.BlockSpec((H, 32), lambda i: (0, 0)),
            pl.BlockSpec((8, 32), lambda i: (0, 0)),
            pl.BlockSpec((32, DOUT), lambda i: (0, 0)),
            pl.BlockSpec((8, DOUT), lambda i: (0, 0)),
        ],
        out_specs=pl.BlockSpec((G, DOUT), lambda i: (0, 0)),
        out_shape=jax.ShapeDtypeStruct((G, DOUT), jnp.float32),
        scratch_shapes=[pltpu.VMEM((G, H), jnp.float32)],
    )(
        parts, parts, hx, rootW, jnp.broadcast_to(bias, (8, H)), batch3,
        f1w, jnp.broadcast_to(f1b, (8, 32)),
        f2w, jnp.broadcast_to(f2b, (8, DOUT)),
    )


def kernel(x, edge_index, edge_attr, batch, nfc_W, nfc_b,
           efc1_W1, efc1_b1, efc1_W2, efc1_b2, gc1_root, gc1_bias,
           efc2_W1, efc2_b1, efc2_W2, efc2_b2, gc2_root, gc2_bias,
           fc1_W, fc1_b, fc2_W, fc2_b):
    src3 = edge_index[0].astype(jnp.int32).reshape(NW, NCH, CH)
    dst3 = edge_index[1].astype(jnp.int32).reshape(NW, NCH, CH)
    zrow = jnp.zeros((NROW, H), jnp.float32)
    batch3 = batch.astype(jnp.int32).reshape(N // BN, 1, BN)
    eat = edge_attr.T

    hx0 = _nfc(x, nfc_W, nfc_b)

    hxs1 = _gather(hx0, src3)
    msg1 = _msg(eat, hxs1, efc1_W1, efc1_b1, efc1_W2, efc1_b2)
    parts1 = _scatter_add(msg1, dst3, zrow)
    hx1 = _combine(parts1, hx0, gc1_root, gc1_bias)

    hxs2 = _gather(hx1, src3)
    msg2 = _msg(eat, hxs2, efc2_W1, efc2_b1, efc2_W2, efc2_b2)
    parts2 = _scatter_add(msg2, dst3, zrow)
    return _final(parts2, hx1, gc2_root, gc2_bias, batch3,
                  fc1_W, fc1_b, fc2_W, fc2_b)


# speedup vs baseline: 3.8796x; 1.0108x over previous
import functools

import jax
import jax.numpy as jnp
from jax import lax
from jax.experimental import pallas as pl
from jax.experimental.pallas import tpu as pltpu
from jax.experimental.pallas import tpu_sc as plsc

N = 8192
E = 32768
G = 256
DN = 128
DE = 16
H = 64
DOUT = 16

NC = 2
NS = 16
NW = NC * NS
EW = E // NW
CH = 128
NCH = EW // CH
NROW = N // NS

BM = 1024
BN = 1024


def _lrelu(v):
    return jnp.where(v >= 0, v, 0.01 * v)


def _nfc_body(x_ref, w_ref, b_ref, o_ref):
    o_ref[...] = _lrelu(
        jnp.dot(x_ref[...], w_ref[...], preferred_element_type=jnp.float32)
        + b_ref[0:1]
    )


def _nfc(x, W, b):
    n, d = x.shape
    h = W.shape[1]
    bm = 2048
    return pl.pallas_call(
        _nfc_body,
        grid=(n // bm,),
        in_specs=[
            pl.BlockSpec((bm, d), lambda i: (i, 0)),
            pl.BlockSpec((d, h), lambda i: (0, 0)),
            pl.BlockSpec((8, h), lambda i: (0, 0)),
        ],
        out_specs=pl.BlockSpec((bm, h), lambda i: (i, 0)),
        out_shape=jax.ShapeDtypeStruct((n, h), jnp.float32),
    )(x, W, jnp.broadcast_to(b, (8, h)))


def _msg_body(eat_ref, hxs_ref, w1t_ref, b1c_ref, w2t_ref, b2rt_ref, o_ref):
    h1t = jnp.maximum(
        jnp.dot(w1t_ref[...], eat_ref[...], preferred_element_type=jnp.float32)
        + b1c_ref[:, 0:1],
        0.0,
    )
    h1tb = h1t.astype(jnp.bfloat16)
    hxst = hxs_ref[...].T
    acc0 = jnp.dot(b2rt_ref[...], hxst, preferred_element_type=jnp.float32)
    acc1 = jnp.zeros_like(acc0)
    NG = 8
    IPG = H // NG
    ROWS = IPG * H
    for g in range(NG):
        tg = jnp.dot(
            w2t_ref[g * ROWS : (g + 1) * ROWS, :], h1tb,
            preferred_element_type=jnp.float32,
        )
        for i2 in range(IPG):
            i = g * IPG + i2
            term = hxst[i : i + 1, :] * tg[i2 * H : (i2 + 1) * H, :]
            if i2 % 2 == 0:
                acc0 = acc0 + term
            else:
                acc1 = acc1 + term
    o_ref[...] = (acc0 + acc1).T


def _msg(edge_attr_t, hxs, W1, b1, W2, b2):
    return pl.pallas_call(
        _msg_body,
        grid=(E // BM,),
        in_specs=[
            pl.BlockSpec((DE, BM), lambda i: (0, i)),
            pl.BlockSpec((BM, H), lambda i: (i, 0)),
            pl.BlockSpec((H, DE), lambda i: (0, 0)),
            pl.BlockSpec((H, 8), lambda i: (0, 0)),
            pl.BlockSpec((H * H, H), lambda i: (0, 0)),
            pl.BlockSpec((H, H), lambda i: (0, 0)),
        ],
        out_specs=pl.BlockSpec((BM, H), lambda i: (i, 0)),
        out_shape=jax.ShapeDtypeStruct((E, H), jnp.float32),
    )(
        edge_attr_t,
        hxs,
        W1.T,
        jnp.broadcast_to(b1[:, None], (H, 8)),
        W2.T.astype(jnp.bfloat16),
        b2.reshape(H, H).T,
    )


def _gather(table, idx3):
    mesh = plsc.VectorSubcoreMesh(core_axis_name="c", subcore_axis_name="s")

    @functools.partial(
        pl.kernel,
        mesh=mesh,
        out_type=jax.ShapeDtypeStruct((E, H), jnp.float32),
        scratch_types=[
            pltpu.VMEM((NCH, CH), jnp.int32),
            pltpu.VMEM((EW, H), jnp.float32),
            pltpu.SemaphoreType.DMA,
        ],
        compiler_params=pltpu.CompilerParams(use_tc_tiling_on_sc=False),
    )
    def k(table_hbm, idx_hbm, out_hbm, idx_v, rows_v, sem):
        cid = lax.axis_index("c")
        sid = lax.axis_index("s")
        wid = cid * NS + sid
        pltpu.sync_copy(idx_hbm.at[wid], idx_v)
        cps = []
        for j in range(NCH):
            cps.append(
                pltpu.async_copy(
                    table_hbm.at[idx_v.at[j]],
                    rows_v.at[pl.ds(j * CH, CH)],
                    sem,
                )
            )
        for c in cps:
            c.wait()
        pltpu.sync_copy(rows_v, out_hbm.at[pl.ds(wid * EW, EW)])

    return k(table, idx3)


def _scatter_add(msg, dst3, zrow):
    mesh = plsc.VectorSubcoreMesh(core_axis_name="c", subcore_axis_name="s")

    @functools.partial(
        pl.kernel,
        mesh=mesh,
        out_type=jax.ShapeDtypeStruct((NC * N, H), jnp.float32),
        scratch_types=[
            pltpu.VMEM((NCH, CH), jnp.int32),
            pltpu.VMEM((EW, H), jnp.float32),
            pltpu.VMEM_SHARED((N, H), jnp.float32),
            pltpu.SemaphoreType.DMA,
            pltpu.SemaphoreType.DMA,
        ],
        compiler_params=pltpu.CompilerParams(use_tc_tiling_on_sc=False),
    )
    def k(msg_hbm, dst_hbm, z_hbm, out_hbm, idx_v, rows_v, acc_sh, sem, sem2):
        cid = lax.axis_index("c")
        sid = lax.axis_index("s")
        wid = cid * NS + sid
        lds = [
            pltpu.async_copy(z_hbm, acc_sh.at[pl.ds(sid * NROW, NROW)], sem),
            pltpu.async_copy(dst_hbm.at[wid], idx_v, sem),
            pltpu.async_copy(msg_hbm.at[pl.ds(wid * EW, EW)], rows_v, sem),
        ]
        for c in lds:
            c.wait()
        plsc.subcore_barrier()
        adds = []
        for j in range(NCH):
            adds.append(
                pltpu.async_copy(
                    rows_v.at[pl.ds(j * CH, CH)],
                    acc_sh.at[idx_v.at[j]],
                    sem2,
                    add=True,
                )
            )
        for c in adds:
            c.wait()
        plsc.subcore_barrier()
        pltpu.sync_copy(
            acc_sh.at[pl.ds(sid * NROW, NROW)],
            out_hbm.at[pl.ds(cid * N + sid * NROW, NROW)],
        )

    return k(msg, dst3, zrow)


def _combine_body(p0_ref, p1_ref, hx_ref, w_ref, b_ref, o_ref):
    o_ref[...] = _lrelu(
        p0_ref[...]
        + p1_ref[...]
        + jnp.dot(hx_ref[...], w_ref[...], preferred_element_type=jnp.float32)
        + b_ref[0:1]
    )


def _combine(parts, hx, rootW, bias):
    bm = 2048
    nb = N // bm
    return pl.pallas_call(
        _combine_body,
        grid=(nb,),
        in_specs=[
            pl.BlockSpec((bm, H), lambda i: (i, 0)),
            pl.BlockSpec((bm, H), lambda i: (i + nb, 0)),
            pl.BlockSpec((bm, H), lambda i: (i, 0)),
            pl.BlockSpec((H, H), lambda i: (0, 0)),
            pl.BlockSpec((8, H), lambda i: (0, 0)),
        ],
        out_specs=pl.BlockSpec((bm, H), lambda i: (i, 0)),
        out_shape=jax.ShapeDtypeStruct((N, H), jnp.float32),
    )(parts, parts, hx, rootW, jnp.broadcast_to(bias, (8, H)))


def _final_body(
    p0_ref, p1_ref, hx_ref, root_ref, gb_ref, batch_ref,
    f1w_ref, f1b_ref, f2w_ref, f2b_ref, o_ref, hg_ref
):
    i = pl.program_id(0)
    hx2 = _lrelu(
        p0_ref[...]
        + p1_ref[...]
        + jnp.dot(hx_ref[...], root_ref[...], preferred_element_type=jnp.float32)
        + gb_ref[0:1]
    )
    b = batch_ref[0]
    gid = lax.broadcasted_iota(jnp.int32, (G, BN), 0)
    m = (b == gid).astype(jnp.float32)
    part = jnp.dot(m, hx2, preferred_element_type=jnp.float32)

    @pl.when(i == 0)
    def _():
        hg_ref[...] = part

    @pl.when(i > 0)
    def _():
        hg_ref[...] = hg_ref[...] + part

    @pl.when(i == pl.num_programs(0) - 1)
    def _():
        hg = _lrelu(
            jnp.dot(hg_ref[...], f1w_ref[...], preferred_element_type=jnp.float32)
            + f1b_ref[0:1]
        )
        o_ref[...] = (
            jnp.dot(hg, f2w_ref[...], preferred_element_type=jnp.float32)
            + f2b_ref[0:1]
        )


def _final(parts, hx, rootW, bias, batch3, f1w, f1b, f2w, f2b):
    nb = N // BN
    return pl.pallas_call(
        _final_body,
        grid=(nb,),
        in_specs=[
            pl.BlockSpec((BN, H), lambda i: (i, 0)),
            pl.BlockSpec((BN, H), lambda i: (i + nb, 0)),
            pl.BlockSpec((BN, H), lambda i: (i, 0)),
            pl.BlockSpec((H, H), lambda i: (0, 0)),
            pl.BlockSpec((8, H), lambda i: (0, 0)),
            pl.BlockSpec((1, 1, BN), lambda i: (i, 0, 0)),
            pl.BlockSpec((H, 32), lambda i: (0, 0)),
            pl.BlockSpec((8, 32), lambda i: (0, 0)),
            pl.BlockSpec((32, DOUT), lambda i: (0, 0)),
            pl.BlockSpec((8, DOUT), lambda i: (0, 0)),
        ],
        out_specs=pl.BlockSpec((G, DOUT), lambda i: (0, 0)),
        out_shape=jax.ShapeDtypeStruct((G, DOUT), jnp.float32),
        scratch_shapes=[pltpu.VMEM((G, H), jnp.float32)],
    )(
        parts, parts, hx, rootW, jnp.broadcast_to(bias, (8, H)), batch3,
        f1w, jnp.broadcast_to(f1b, (8, 32)),
        f2w, jnp.broadcast_to(f2b, (8, DOUT)),
    )


def kernel(x, edge_index, edge_attr, batch, nfc_W, nfc_b,
           efc1_W1, efc1_b1, efc1_W2, efc1_b2, gc1_root, gc1_bias,
           efc2_W1, efc2_b1, efc2_W2, efc2_b2, gc2_root, gc2_bias,
           fc1_W, fc1_b, fc2_W, fc2_b):
    src3 = edge_index[0].astype(jnp.int32).reshape(NW, NCH, CH)
    dst3 = edge_index[1].astype(jnp.int32).reshape(NW, NCH, CH)
    zrow = jnp.zeros((NROW, H), jnp.float32)
    batch3 = batch.astype(jnp.int32).reshape(N // BN, 1, BN)
    eat = edge_attr.T

    hx0 = _nfc(x, nfc_W, nfc_b)

    hxs1 = _gather(hx0, src3)
    msg1 = _msg(eat, hxs1, efc1_W1, efc1_b1, efc1_W2, efc1_b2)
    parts1 = _scatter_add(msg1, dst3, zrow)
    hx1 = _combine(parts1, hx0, gc1_root, gc1_bias)

    hxs2 = _gather(hx1, src3)
    msg2 = _msg(eat, hxs2, efc2_W1, efc2_b1, efc2_W2, efc2_b2)
    parts2 = _scatter_add(msg2, dst3, zrow)
    return _final(parts2, hx1, gc2_root, gc2_bias, batch3,
                  fc1_W, fc1_b, fc2_W, fc2_b)


# speedup vs baseline: 4.0523x; 1.0445x over previous
import functools

import jax
import jax.numpy as jnp
from jax import lax
from jax.experimental import pallas as pl
from jax.experimental.pallas import tpu as pltpu
from jax.experimental.pallas import tpu_sc as plsc

N = 8192
E = 32768
G = 256
DN = 128
DE = 16
H = 64
HP = 128
DOUT = 16

NC = 2
NS = 16
NW = NC * NS
EW = E // NW
CH = 128
NCH = EW // CH
NP = 2
EP = EW // NP
CP = NCH // NP
NROW = N // NS
HN = N // NC
ES = E // NS
NCH2 = ES // CH
NP2 = ES // EP
ZR = HN // NS

BM = 1024
BN = 1024


def _lrelu(v):
    return jnp.where(v >= 0, v, 0.01 * v)


def _pad128(v):
    return jnp.concatenate([v, jnp.zeros_like(v)], axis=1)


def _nfc_body(x_ref, w_ref, b_ref, o_ref):
    o_ref[...] = _pad128(_lrelu(
        jnp.dot(x_ref[...], w_ref[...], preferred_element_type=jnp.float32)
        + b_ref[0:1]
    ))


def _nfc(x, W, b):
    n, d = x.shape
    h = W.shape[1]
    bm = 2048
    return pl.pallas_call(
        _nfc_body,
        grid=(n // bm,),
        in_specs=[
            pl.BlockSpec((bm, d), lambda i: (i, 0)),
            pl.BlockSpec((d, h), lambda i: (0, 0)),
            pl.BlockSpec((8, h), lambda i: (0, 0)),
        ],
        out_specs=pl.BlockSpec((bm, HP), lambda i: (i, 0)),
        out_shape=jax.ShapeDtypeStruct((n, HP), jnp.float32),
    )(x, W, jnp.broadcast_to(b, (8, h)))


def _msg_body(eat_ref, hxs_ref, w1t_ref, b1c_ref, w2t_ref, b2rt_ref, o_ref):
    h1t = jnp.maximum(
        jnp.dot(w1t_ref[...], eat_ref[...], preferred_element_type=jnp.float32)
        + b1c_ref[:, 0:1],
        0.0,
    )
    h1tb = h1t.astype(jnp.bfloat16)
    hxst = hxs_ref[...][:, 0:H].T
    acc0 = jnp.dot(b2rt_ref[...], hxst, preferred_element_type=jnp.float32)
    acc1 = jnp.zeros_like(acc0)
    NG = 8
    IPG = H // NG
    ROWS = IPG * H
    for g in range(NG):
        tg = jnp.dot(
            w2t_ref[g * ROWS : (g + 1) * ROWS, :], h1tb,
            preferred_element_type=jnp.float32,
        )
        for i2 in range(IPG):
            i = g * IPG + i2
            term = hxst[i : i + 1, :] * tg[i2 * H : (i2 + 1) * H, :]
            if i2 % 2 == 0:
                acc0 = acc0 + term
            else:
                acc1 = acc1 + term
    o_ref[...] = _pad128((acc0 + acc1).T)


def _msg(edge_attr_t, hxs, W1, b1, W2, b2):
    return pl.pallas_call(
        _msg_body,
        grid=(E // BM,),
        in_specs=[
            pl.BlockSpec((DE, BM), lambda i: (0, i)),
            pl.BlockSpec((BM, HP), lambda i: (i, 0)),
            pl.BlockSpec((H, DE), lambda i: (0, 0)),
            pl.BlockSpec((H, 8), lambda i: (0, 0)),
            pl.BlockSpec((H * H, H), lambda i: (0, 0)),
            pl.BlockSpec((H, H), lambda i: (0, 0)),
        ],
        out_specs=pl.BlockSpec((BM, HP), lambda i: (i, 0)),
        out_shape=jax.ShapeDtypeStruct((E, HP), jnp.float32),
    )(
        edge_attr_t,
        hxs,
        W1.T,
        jnp.broadcast_to(b1[:, None], (H, 8)),
        W2.T.astype(jnp.bfloat16),
        b2.reshape(H, H).T,
    )


def _gather(table, idx3):
    mesh = plsc.VectorSubcoreMesh(core_axis_name="c", subcore_axis_name="s")

    @functools.partial(
        pl.kernel,
        mesh=mesh,
        out_type=jax.ShapeDtypeStruct((E, HP), jnp.float32),
        scratch_types=[
            pltpu.VMEM((NCH, CH), jnp.int32),
            pltpu.VMEM((EP, HP), jnp.float32),
            pltpu.SemaphoreType.DMA,
        ],
    )
    def k(table_hbm, idx_hbm, out_hbm, idx_v, rows_v, sem):
        cid = lax.axis_index("c")
        sid = lax.axis_index("s")
        wid = cid * NS + sid
        pltpu.sync_copy(idx_hbm.at[wid], idx_v)
        for p in range(NP):
            cps = []
            for q in range(CP):
                j = p * CP + q
                cps.append(
                    pltpu.async_copy(
                        table_hbm.at[idx_v.at[j]],
                        rows_v.at[pl.ds(q * CH, CH)],
                        sem,
                    )
                )
            for c in cps:
                c.wait()
            pltpu.sync_copy(
                rows_v, out_hbm.at[pl.ds(wid * EW + p * EP, EP)]
            )

    return k(table, idx3)


def _scatter_add(msg, dst2, zrow):
    mesh = plsc.VectorSubcoreMesh(core_axis_name="c", subcore_axis_name="s")

    @functools.partial(
        pl.kernel,
        mesh=mesh,
        out_type=jax.ShapeDtypeStruct((N, HP), jnp.float32),
        scratch_types=[
            pltpu.VMEM((NCH2, CH), jnp.int32),
            pltpu.VMEM((EP, HP), jnp.float32),
            pltpu.VMEM_SHARED((HN + 8, HP), jnp.float32),
            pltpu.SemaphoreType.DMA,
            pltpu.SemaphoreType.DMA,
        ],
    )
    def k(msg_hbm, dst_hbm, z_hbm, out_hbm, idx_v, rows_v, acc_sh, sem, sem2):
        cid = lax.axis_index("c")
        sid = lax.axis_index("s")
        lds = [
            pltpu.async_copy(z_hbm, acc_sh.at[pl.ds(sid * ZR, ZR)], sem),
            pltpu.async_copy(dst_hbm.at[sid], idx_v, sem),
        ]
        for c in lds:
            c.wait()
        base = cid * HN
        for r in range(NCH2):
            for kk in range(CH // 16):
                v = idx_v[r, pl.ds(kk * 16, 16)] - base
                ok = (v >= 0) & (v < HN)
                idx_v[r, pl.ds(kk * 16, 16)] = jnp.where(ok, v, HN)
        plsc.subcore_barrier()
        for p in range(NP2):
            pltpu.sync_copy(
                msg_hbm.at[pl.ds(sid * ES + p * EP, EP)], rows_v
            )
            adds = []
            for q in range(CP):
                j = p * CP + q
                adds.append(
                    pltpu.async_copy(
                        rows_v.at[pl.ds(q * CH, CH)],
                        acc_sh.at[idx_v.at[j]],
                        sem2,
                        add=True,
                    )
                )
            for c in adds:
                c.wait()
        plsc.subcore_barrier()
        pltpu.sync_copy(
            acc_sh.at[pl.ds(sid * ZR, ZR)],
            out_hbm.at[pl.ds(cid * HN + sid * ZR, ZR)],
        )

    return k(msg, dst2, zrow)


def _combine_body(p_ref, hx_ref, w_ref, b_ref, o_ref):
    o_ref[...] = _pad128(_lrelu(
        p_ref[...][:, 0:H]
        + jnp.dot(hx_ref[...][:, 0:H], w_ref[...],
                  preferred_element_type=jnp.float32)
        + b_ref[0:1]
    ))


def _combine(agg, hx, rootW, bias):
    bm = 2048
    nb = N // bm
    return pl.pallas_call(
        _combine_body,
        grid=(nb,),
        in_specs=[
            pl.BlockSpec((bm, HP), lambda i: (i, 0)),
            pl.BlockSpec((bm, HP), lambda i: (i, 0)),
            pl.BlockSpec((H, H), lambda i: (0, 0)),
            pl.BlockSpec((8, H), lambda i: (0, 0)),
        ],
        out_specs=pl.BlockSpec((bm, HP), lambda i: (i, 0)),
        out_shape=jax.ShapeDtypeStruct((N, HP), jnp.float32),
    )(agg, hx, rootW, jnp.broadcast_to(bias, (8, H)))


def _final_body(
    p_ref, hx_ref, root_ref, gb_ref, batch_ref,
    f1w_ref, f1b_ref, f2w_ref, f2b_ref, o_ref, hg_ref
):
    i = pl.program_id(0)
    hx2 = _lrelu(
        p_ref[...][:, 0:H]
        + jnp.dot(hx_ref[...][:, 0:H], root_ref[...],
                  preferred_element_type=jnp.float32)
        + gb_ref[0:1]
    )
    b = batch_ref[0]
    gid = lax.broadcasted_iota(jnp.int32, (G, BN), 0)
    m = (b == gid).astype(jnp.float32)
    part = jnp.dot(m, hx2, preferred_element_type=jnp.float32)

    @pl.when(i == 0)
    def _():
        hg_ref[...] = part

    @pl.when(i > 0)
    def _():
        hg_ref[...] = hg_ref[...] + part

    @pl.when(i == pl.num_programs(0) - 1)
    def _():
        hg = _lrelu(
            jnp.dot(hg_ref[...], f1w_ref[...], preferred_element_type=jnp.float32)
            + f1b_ref[0:1]
        )
        o_ref[...] = (
            jnp.dot(hg, f2w_ref[...], preferred_element_type=jnp.float32)
            + f2b_ref[0:1]
        )


def _final(agg, hx, rootW, bias, batch3, f1w, f1b, f2w, f2b):
    nb = N // BN
    return pl.pallas_call(
        _final_body,
        grid=(nb,),
        in_specs=[
            pl.BlockSpec((BN, HP), lambda i: (i, 0)),
            pl.BlockSpec((BN, HP), lambda i: (i, 0)),
            pl.BlockSpec((H, H), lambda i: (0, 0)),
            pl.BlockSpec((8, H), lambda i: (0, 0)),
            pl.BlockSpec((1, 1, BN), lambda i: (i, 0, 0)),
            pl.BlockSpec((H, 32), lambda i: (0, 0)),
            pl.BlockSpec((8, 32), lambda i: (0, 0)),
            pl.BlockSpec((32, DOUT), lambda i: (0, 0)),
            pl.BlockSpec((8, DOUT), lambda i: (0, 0)),
        ],
        out_specs=pl.BlockSpec((G, DOUT), lambda i: (0, 0)),
        out_shape=jax.ShapeDtypeStruct((G, DOUT), jnp.float32),
        scratch_shapes=[pltpu.VMEM((G, H), jnp.float32)],
    )(
        agg, hx, rootW, jnp.broadcast_to(bias, (8, H)), batch3,
        f1w, jnp.broadcast_to(f1b, (8, 32)),
        f2w, jnp.broadcast_to(f2b, (8, DOUT)),
    )


def kernel(x, edge_index, edge_attr, batch, nfc_W, nfc_b,
           efc1_W1, efc1_b1, efc1_W2, efc1_b2, gc1_root, gc1_bias,
           efc2_W1, efc2_b1, efc2_W2, efc2_b2, gc2_root, gc2_bias,
           fc1_W, fc1_b, fc2_W, fc2_b):
    src3 = edge_index[0].astype(jnp.int32).reshape(NW, NCH, CH)
    dst2 = edge_index[1].astype(jnp.int32).reshape(NS, NCH2, CH)
    zrow = jnp.zeros((ZR, HP), jnp.float32)
    batch3 = batch.astype(jnp.int32).reshape(N // BN, 1, BN)
    eat = edge_attr.T

    hx0 = _nfc(x, nfc_W, nfc_b)

    hxs1 = _gather(hx0, src3)
    msg1 = _msg(eat, hxs1, efc1_W1, efc1_b1, efc1_W2, efc1_b2)
    agg1 = _scatter_add(msg1, dst2, zrow)
    hx1 = _combine(agg1, hx0, gc1_root, gc1_bias)

    hxs2 = _gather(hx1, src3)
    msg2 = _msg(eat, hxs2, efc2_W1, efc2_b1, efc2_W2, efc2_b2)
    agg2 = _scatter_add(msg2, dst2, zrow)
    return _final(agg2, hx1, gc2_root, gc2_bias, batch3,
                  fc1_W, fc1_b, fc2_W, fc2_b)


# speedup vs baseline: 4.1499x; 1.0241x over previous
import functools

import jax
import jax.numpy as jnp
from jax import lax
from jax.experimental import pallas as pl
from jax.experimental.pallas import tpu as pltpu
from jax.experimental.pallas import tpu_sc as plsc

N = 8192
E = 32768
G = 256
DN = 128
DE = 16
H = 64
HP = 128
DOUT = 16

NC = 2
NS = 16
NW = NC * NS
EW = E // NW
CH = 128
NCH = EW // CH
NP = 2
EP = EW // NP
CP = NCH // NP
NROW = N // NS
HN = N // NC
ES = E // NS
NCH2 = ES // CH
NP2 = ES // EP
ZR = HN // NS

BM = 1024
BN = 1024


def _lrelu(v):
    return jnp.where(v >= 0, v, 0.01 * v)


def _pad128(v):
    return jnp.concatenate([v, jnp.zeros_like(v)], axis=1)


def _nfc_body(x_ref, w_ref, b_ref, o_ref):
    o_ref[:, 0:H] = _lrelu(
        jnp.dot(x_ref[...], w_ref[...], preferred_element_type=jnp.float32)
        + b_ref[0:1]
    )


def _nfc(x, W, b):
    n, d = x.shape
    h = W.shape[1]
    bm = 2048
    return pl.pallas_call(
        _nfc_body,
        grid=(n // bm,),
        in_specs=[
            pl.BlockSpec((bm, d), lambda i: (i, 0)),
            pl.BlockSpec((d, h), lambda i: (0, 0)),
            pl.BlockSpec((8, h), lambda i: (0, 0)),
        ],
        out_specs=pl.BlockSpec((bm, HP), lambda i: (i, 0)),
        out_shape=jax.ShapeDtypeStruct((n, HP), jnp.float32),
    )(x, W, jnp.broadcast_to(b, (8, h)))


def _msg_body(eat_ref, hxs_ref, w1t_ref, b1c_ref, w2t_ref, b2rt_ref, o_ref):
    h1t = jnp.maximum(
        jnp.dot(w1t_ref[...], eat_ref[...], preferred_element_type=jnp.float32)
        + b1c_ref[:, 0:1],
        0.0,
    )
    h1tb = h1t.astype(jnp.bfloat16)
    hxst = hxs_ref[...][:, 0:H].T
    acc0 = jnp.dot(b2rt_ref[...], hxst, preferred_element_type=jnp.float32)
    acc1 = jnp.zeros_like(acc0)
    NG = 8
    IPG = H // NG
    ROWS = IPG * H
    for g in range(NG):
        tg = jnp.dot(
            w2t_ref[g * ROWS : (g + 1) * ROWS, :], h1tb,
            preferred_element_type=jnp.float32,
        )
        for i2 in range(IPG):
            i = g * IPG + i2
            term = hxst[i : i + 1, :] * tg[i2 * H : (i2 + 1) * H, :]
            if i2 % 2 == 0:
                acc0 = acc0 + term
            else:
                acc1 = acc1 + term
    o_ref[:, 0:H] = (acc0 + acc1).T


def _msg(edge_attr_t, hxs, W1, b1, W2, b2):
    return pl.pallas_call(
        _msg_body,
        grid=(E // BM,),
        in_specs=[
            pl.BlockSpec((DE, BM), lambda i: (0, i)),
            pl.BlockSpec((BM, HP), lambda i: (i, 0)),
            pl.BlockSpec((H, DE), lambda i: (0, 0)),
            pl.BlockSpec((H, 8), lambda i: (0, 0)),
            pl.BlockSpec((H * H, H), lambda i: (0, 0)),
            pl.BlockSpec((H, H), lambda i: (0, 0)),
        ],
        out_specs=pl.BlockSpec((BM, HP), lambda i: (i, 0)),
        out_shape=jax.ShapeDtypeStruct((E, HP), jnp.float32),
    )(
        edge_attr_t,
        hxs,
        W1.T,
        jnp.broadcast_to(b1[:, None], (H, 8)),
        W2.T.astype(jnp.bfloat16),
        b2.reshape(H, H).T,
    )


def _gather(table, idx3):
    mesh = plsc.VectorSubcoreMesh(core_axis_name="c", subcore_axis_name="s")

    GP = 256
    GNP = EW // GP
    GCP = GP // CH

    @functools.partial(
        pl.kernel,
        mesh=mesh,
        out_type=jax.ShapeDtypeStruct((E, HP), jnp.float32),
        scratch_types=[
            pltpu.VMEM((NCH, CH), jnp.int32),
            pltpu.VMEM((GP, HP), jnp.float32),
            pltpu.VMEM((GP, HP), jnp.float32),
            pltpu.SemaphoreType.DMA,
            pltpu.SemaphoreType.DMA,
            pltpu.SemaphoreType.DMA,
        ],
    )
    def k(table_hbm, idx_hbm, out_hbm, idx_v, rva, rvb, semg, semw0, semw1):
        cid = lax.axis_index("c")
        sid = lax.axis_index("s")
        wid = cid * NS + sid
        pltpu.sync_copy(idx_hbm.at[wid], idx_v)
        bufs = [rva, rvb]
        wsems = [semw0, semw1]
        wbh = [None] * GNP
        for p in range(GNP):
            b = p % 2
            if p >= 2:
                wbh[p - 2].wait()
            cps = []
            for q in range(GCP):
                j = p * GCP + q
                cps.append(
                    pltpu.async_copy(
                        table_hbm.at[idx_v.at[j]],
                        bufs[b].at[pl.ds(q * CH, CH)],
                        semg,
                    )
                )
            for c in cps:
                c.wait()
            wbh[p] = pltpu.async_copy(
                bufs[b], out_hbm.at[pl.ds(wid * EW + p * GP, GP)], wsems[b]
            )
        for p in range(GNP - 2, GNP):
            wbh[p].wait()

    return k(table, idx3)


def _scatter_add(msg, dst2, zrow):
    mesh = plsc.VectorSubcoreMesh(core_axis_name="c", subcore_axis_name="s")

    @functools.partial(
        pl.kernel,
        mesh=mesh,
        out_type=jax.ShapeDtypeStruct((N, HP), jnp.float32),
        scratch_types=[
            pltpu.VMEM((NCH2, CH), jnp.int32),
            pltpu.VMEM((256, HP), jnp.float32),
            pltpu.VMEM((256, HP), jnp.float32),
            pltpu.VMEM_SHARED((HN + 8, HP), jnp.float32),
            pltpu.SemaphoreType.DMA,
            pltpu.SemaphoreType.DMA,
            pltpu.SemaphoreType.DMA,
            pltpu.SemaphoreType.DMA,
        ],
    )
    def k(msg_hbm, dst_hbm, z_hbm, out_hbm, idx_v, rva, rvb, acc_sh,
          sem, sem2, seml0, seml1):
        cid = lax.axis_index("c")
        sid = lax.axis_index("s")
        lds = [
            pltpu.async_copy(z_hbm, acc_sh.at[pl.ds(sid * ZR, ZR)], sem),
            pltpu.async_copy(dst_hbm.at[sid], idx_v, sem),
        ]
        for c in lds:
            c.wait()
        base = cid * HN
        for r in range(NCH2):
            for kk in range(CH // 16):
                v = idx_v[r, pl.ds(kk * 16, 16)] - base
                ok = (v >= 0) & (v < HN)
                idx_v[r, pl.ds(kk * 16, 16)] = jnp.where(ok, v, HN)
        plsc.subcore_barrier()
        SP = 256
        SNP = ES // SP
        SCP = SP // CH
        bufs = [rva, rvb]
        lsems = [seml0, seml1]
        lh = [None] * SNP
        lh[0] = pltpu.async_copy(
            msg_hbm.at[pl.ds(sid * ES, SP)], bufs[0], lsems[0]
        )
        for p in range(SNP):
            b = p % 2
            if p + 1 < SNP:
                lh[p + 1] = pltpu.async_copy(
                    msg_hbm.at[pl.ds(sid * ES + (p + 1) * SP, SP)],
                    bufs[(p + 1) % 2],
                    lsems[(p + 1) % 2],
                )
            lh[p].wait()
            adds = []
            for q in range(SCP):
                j = p * SCP + q
                adds.append(
                    pltpu.async_copy(
                        bufs[b].at[pl.ds(q * CH, CH)],
                        acc_sh.at[idx_v.at[j]],
                        sem2,
                        add=True,
                    )
                )
            for c in adds:
                c.wait()
        plsc.subcore_barrier()
        pltpu.sync_copy(
            acc_sh.at[pl.ds(sid * ZR, ZR)],
            out_hbm.at[pl.ds(cid * HN + sid * ZR, ZR)],
        )

    return k(msg, dst2, zrow)


def _combine_body(p_ref, hx_ref, w_ref, b_ref, o_ref):
    o_ref[:, 0:H] = _lrelu(
        p_ref[...][:, 0:H]
        + jnp.dot(hx_ref[...][:, 0:H], w_ref[...],
                  preferred_element_type=jnp.float32)
        + b_ref[0:1]
    )


def _combine(agg, hx, rootW, bias):
    bm = 2048
    nb = N // bm
    return pl.pallas_call(
        _combine_body,
        grid=(nb,),
        in_specs=[
            pl.BlockSpec((bm, HP), lambda i: (i, 0)),
            pl.BlockSpec((bm, HP), lambda i: (i, 0)),
            pl.BlockSpec((H, H), lambda i: (0, 0)),
            pl.BlockSpec((8, H), lambda i: (0, 0)),
        ],
        out_specs=pl.BlockSpec((bm, HP), lambda i: (i, 0)),
        out_shape=jax.ShapeDtypeStruct((N, HP), jnp.float32),
    )(agg, hx, rootW, jnp.broadcast_to(bias, (8, H)))


def _final_body(
    p_ref, hx_ref, root_ref, gb_ref, batch_ref,
    f1w_ref, f1b_ref, f2w_ref, f2b_ref, o_ref, hg_ref
):
    i = pl.program_id(0)
    hx2 = _lrelu(
        p_ref[...][:, 0:H]
        + jnp.dot(hx_ref[...][:, 0:H], root_ref[...],
                  preferred_element_type=jnp.float32)
        + gb_ref[0:1]
    )
    b = batch_ref[0]
    gid = lax.broadcasted_iota(jnp.int32, (G, BN), 0)
    m = (b == gid).astype(jnp.float32)
    part = jnp.dot(m, hx2, preferred_element_type=jnp.float32)

    @pl.when(i == 0)
    def _():
        hg_ref[...] = part

    @pl.when(i > 0)
    def _():
        hg_ref[...] = hg_ref[...] + part

    @pl.when(i == pl.num_programs(0) - 1)
    def _():
        hg = _lrelu(
            jnp.dot(hg_ref[...], f1w_ref[...], preferred_element_type=jnp.float32)
            + f1b_ref[0:1]
        )
        o_ref[...] = (
            jnp.dot(hg, f2w_ref[...], preferred_element_type=jnp.float32)
            + f2b_ref[0:1]
        )


def _final(agg, hx, rootW, bias, batch3, f1w, f1b, f2w, f2b):
    nb = N // BN
    return pl.pallas_call(
        _final_body,
        grid=(nb,),
        in_specs=[
            pl.BlockSpec((BN, HP), lambda i: (i, 0)),
            pl.BlockSpec((BN, HP), lambda i: (i, 0)),
            pl.BlockSpec((H, H), lambda i: (0, 0)),
            pl.BlockSpec((8, H), lambda i: (0, 0)),
            pl.BlockSpec((1, 1, BN), lambda i: (i, 0, 0)),
            pl.BlockSpec((H, 32), lambda i: (0, 0)),
            pl.BlockSpec((8, 32), lambda i: (0, 0)),
            pl.BlockSpec((32, DOUT), lambda i: (0, 0)),
            pl.BlockSpec((8, DOUT), lambda i: (0, 0)),
        ],
        out_specs=pl.BlockSpec((G, DOUT), lambda i: (0, 0)),
        out_shape=jax.ShapeDtypeStruct((G, DOUT), jnp.float32),
        scratch_shapes=[pltpu.VMEM((G, H), jnp.float32)],
    )(
        agg, hx, rootW, jnp.broadcast_to(bias, (8, H)), batch3,
        f1w, jnp.broadcast_to(f1b, (8, 32)),
        f2w, jnp.broadcast_to(f2b, (8, DOUT)),
    )


def kernel(x, edge_index, edge_attr, batch, nfc_W, nfc_b,
           efc1_W1, efc1_b1, efc1_W2, efc1_b2, gc1_root, gc1_bias,
           efc2_W1, efc2_b1, efc2_W2, efc2_b2, gc2_root, gc2_bias,
           fc1_W, fc1_b, fc2_W, fc2_b):
    src3 = edge_index[0].astype(jnp.int32).reshape(NW, NCH, CH)
    dst2 = edge_index[1].astype(jnp.int32).reshape(NS, NCH2, CH)
    zrow = jnp.zeros((ZR, HP), jnp.float32)
    batch3 = batch.astype(jnp.int32).reshape(N // BN, 1, BN)
    eat = edge_attr.T

    hx0 = _nfc(x, nfc_W, nfc_b)

    hxs1 = _gather(hx0, src3)
    msg1 = _msg(eat, hxs1, efc1_W1, efc1_b1, efc1_W2, efc1_b2)
    agg1 = _scatter_add(msg1, dst2, zrow)
    hx1 = _combine(agg1, hx0, gc1_root, gc1_bias)

    hxs2 = _gather(hx1, src3)
    msg2 = _msg(eat, hxs2, efc2_W1, efc2_b1, efc2_W2, efc2_b2)
    agg2 = _scatter_add(msg2, dst2, zrow)
    return _final(agg2, hx1, gc2_root, gc2_bias, batch3,
                  fc1_W, fc1_b, fc2_W, fc2_b)
